# baseline (speedup 1.0000x reference)
import functools

import jax
import jax.numpy as jnp
from jax import lax
from jax.experimental import pallas as pl
from jax.experimental.pallas import tpu as pltpu
from jax.experimental.pallas import tpu_sc as plsc

B = 8192
D = 2048
E = 16
R = 128
TOKEN_TILE = 256
SLOT_TILE = 256
S = 2 * B + E * SLOT_TILE
NW = 32
PER_W = S // NW
CHUNK = 40
NCHUNK = PER_W // CHUNK
B2P = 2 * B + 512
DUMP = 2 * B
NEG_BIG = -3.0e38


def _encode_route_body(x_ref, wenc_ref, benc_ref, wgate_ref, gamma_ref,
                       enc_ref, wq_ref, idx_ref):
    prec = jax.lax.Precision.DEFAULT
    enc = jax.lax.dot_general(
        x_ref[...], wenc_ref[...], (((1,), (1,)), ((), ())),
        precision=prec, preferred_element_type=jnp.float32)
    enc = enc + benc_ref[...]
    enc_ref[...] = enc

    logits = jax.lax.dot_general(
        enc, wgate_ref[...], (((1,), (1,)), ((), ())),
        precision=prec, preferred_element_type=jnp.float32)

    lane = jax.lax.broadcasted_iota(jnp.int32, logits.shape, 1)
    v0 = jnp.max(logits, axis=1, keepdims=True)
    i0 = jnp.min(jnp.where(logits == v0, lane, E), axis=1, keepdims=True)
    masked = jnp.where(lane == i0, NEG_BIG, logits)
    v1 = jnp.max(masked, axis=1, keepdims=True)
    i1 = jnp.min(jnp.where(masked == v1, lane, E), axis=1, keepdims=True)

    e1 = jnp.exp(v1 - v0)
    denom = 1.0 + e1 + 1e-12
    w0 = 1.0 / denom
    w1 = e1 / denom
    w0 = jnp.where(w0 > 1e-12, w0, 0.0)
    w1 = jnp.where(w1 > 1e-12, w1, 0.0)

    g = gamma_ref[...]
    g0 = jnp.sum(jnp.where(lane == i0, g, 0.0), axis=1, keepdims=True)
    g1 = jnp.sum(jnp.where(lane == i1, g, 0.0), axis=1, keepdims=True)

    t = w0.shape[0]
    wq_ref[...] = jnp.concatenate(
        [w0 + w1, w0 * g0, w1 * g1, jnp.zeros((t, 5), jnp.float32)], axis=1)
    idx_ref[...] = jnp.concatenate([i0, i1], axis=1)


def _encode_route(x, W_enc, b_enc, W_gate, gamma):
    grid = (B // TOKEN_TILE,)
    return pl.pallas_call(
        _encode_route_body,
        grid=grid,
        in_specs=[
            pl.BlockSpec((TOKEN_TILE, D), lambda i: (i, 0)),
            pl.BlockSpec((D, D), lambda i: (0, 0)),
            pl.BlockSpec((1, D), lambda i: (0, 0)),
            pl.BlockSpec((E, D), lambda i: (0, 0)),
            pl.BlockSpec((1, E), lambda i: (0, 0)),
        ],
        out_specs=[
            pl.BlockSpec((TOKEN_TILE, D), lambda i: (i, 0)),
            pl.BlockSpec((TOKEN_TILE, 8), lambda i: (i, 0)),
            pl.BlockSpec((TOKEN_TILE, 2), lambda i: (i, 0)),
        ],
        out_shape=[
            jax.ShapeDtypeStruct((B, D), jnp.float32),
            jax.ShapeDtypeStruct((B, 8), jnp.float32),
            jax.ShapeDtypeStruct((B, 2), jnp.int32),
        ],
        compiler_params=pltpu.CompilerParams(
            dimension_semantics=("arbitrary",)),
    )(x, W_enc, b_enc.reshape(1, D), W_gate, gamma.reshape(1, E))


def _dispatch_plan(idx):
    i32 = jnp.int32
    e_flat = idx.reshape(-1).astype(i32)
    order = jnp.argsort(e_flat, stable=True).astype(i32)
    sorted_e = e_flat[order]
    counts = jnp.bincount(e_flat, length=E).astype(i32)
    padded = ((counts + SLOT_TILE - 1) // SLOT_TILE) * SLOT_TILE
    ends_pad = jnp.cumsum(padded).astype(i32)
    starts_pad = ends_pad - padded
    ends_raw = jnp.cumsum(counts).astype(i32)
    starts_raw = ends_raw - counts

    s_ids = jnp.arange(S, dtype=i32)
    e_of_s = jnp.searchsorted(ends_pad, s_ids, side="right").astype(i32)
    e_of_s = jnp.minimum(e_of_s, E - 1)
    r_of_s = s_ids - starts_pad[e_of_s]
    valid = r_of_s < counts[e_of_s]
    p_of_s = starts_raw[e_of_s] + jnp.minimum(r_of_s,
                                              jnp.maximum(counts[e_of_s] - 1, 0))
    j_of_s = order[p_of_s]
    slot_token = jnp.where(valid, j_of_s // 2, 0).astype(i32)
    slot_out = jnp.where(valid, j_of_s, DUMP).astype(i32)

    tile_start = jnp.arange(S // SLOT_TILE, dtype=i32) * SLOT_TILE
    tile_expert = jnp.searchsorted(ends_pad, tile_start,
                                   side="right").astype(i32)
    tile_expert = jnp.minimum(tile_expert, E - 1)
    return slot_token, slot_out, tile_expert


@functools.lru_cache(maxsize=None)
def _sc_mesh():
    return plsc.VectorSubcoreMesh(core_axis_name="c", subcore_axis_name="s")


@functools.lru_cache(maxsize=None)
def _sc_gather_kernel():
    @functools.partial(
        pl.kernel,
        out_type=jax.ShapeDtypeStruct((S, D), jnp.float32),
        mesh=_sc_mesh(),
        scratch_types=[
            pltpu.VMEM((CHUNK,), jnp.int32),
            pltpu.VMEM((CHUNK, D), jnp.float32),
            pltpu.SemaphoreType.DMA,
        ],
    )
    def body(enc_hbm, tok_hbm, out_hbm, idx_v, rows_v, sem):
        wid = lax.axis_index("s") * 2 + lax.axis_index("c")
        base = wid * PER_W
        for c in range(NCHUNK):
            off = base + c * CHUNK
            pltpu.sync_copy(tok_hbm.at[pl.ds(off, CHUNK)], idx_v)
            pltpu.async_copy(enc_hbm.at[idx_v], rows_v, sem).wait()
            pltpu.sync_copy(rows_v, out_hbm.at[pl.ds(off, CHUNK)])
    return body


def _sc_gather(encoded, slot_token):
    return _sc_gather_kernel()(encoded, slot_token)


@functools.lru_cache(maxsize=None)
def _sc_scatter_kernel():
    @functools.partial(
        pl.kernel,
        out_type=jax.ShapeDtypeStruct((B2P, D), jnp.float32),
        mesh=_sc_mesh(),
        scratch_types=[
            pltpu.VMEM((CHUNK,), jnp.int32),
            pltpu.VMEM((CHUNK, D), jnp.float32),
            pltpu.SemaphoreType.DMA,
        ],
    )
    def body(osort_hbm, dst_hbm, out_hbm, idx_v, rows_v, sem):
        wid = lax.axis_index("s") * 2 + lax.axis_index("c")
        base = wid * PER_W
        for c in range(NCHUNK):
            off = base + c * CHUNK
            pltpu.sync_copy(dst_hbm.at[pl.ds(off, CHUNK)], idx_v)
            pltpu.sync_copy(osort_hbm.at[pl.ds(off, CHUNK)], rows_v)
            pltpu.async_copy(rows_v, out_hbm.at[idx_v], sem).wait()
    return body


def _sc_scatter(o_sorted, slot_out):
    return _sc_scatter_kernel()(o_sorted, slot_out)


def _experts_body(te_ref, x_ref, u_ref, vt_ref, o_ref):
    prec = jax.lax.Precision.DEFAULT
    h = jax.lax.dot_general(
        x_ref[...], u_ref[0], (((1,), (1,)), ((), ())),
        precision=prec, preferred_element_type=jnp.float32)
    h = h * jax.nn.sigmoid(h)
    o_ref[...] = jax.lax.dot_general(
        h, vt_ref[0], (((1,), (0,)), ((), ())),
        precision=prec, preferred_element_type=jnp.float32)


def _experts(x_sorted, tile_expert, U, Vt):
    grid = (S // SLOT_TILE,)
    spec = pltpu.PrefetchScalarGridSpec(
        num_scalar_prefetch=1,
        grid=grid,
        in_specs=[
            pl.BlockSpec((SLOT_TILE, D), lambda i, te: (i, 0)),
            pl.BlockSpec((1, R, D), lambda i, te: (te[i], 0, 0)),
            pl.BlockSpec((1, R, D), lambda i, te: (te[i], 0, 0)),
        ],
        out_specs=pl.BlockSpec((SLOT_TILE, D), lambda i, te: (i, 0)),
    )
    return pl.pallas_call(
        _experts_body,
        grid_spec=spec,
        out_shape=jax.ShapeDtypeStruct((S, D), jnp.float32),
        compiler_params=pltpu.CompilerParams(
            dimension_semantics=("arbitrary",)),
    )(tile_expert, x_sorted, U, Vt)


def _combine_body(enc_ref, wq_ref, ot_ref, y_ref):
    ot = ot_ref[...].reshape(TOKEN_TILE, 2, D)
    wq = wq_ref[...]
    y_ref[...] = (enc_ref[...] * wq[:, 0:1]
                  + wq[:, 1:2] * ot[:, 0, :]
                  + wq[:, 2:3] * ot[:, 1, :])


def _combine(encoded, wq, o_tok):
    grid = (B // TOKEN_TILE,)
    return pl.pallas_call(
        _combine_body,
        grid=grid,
        in_specs=[
            pl.BlockSpec((TOKEN_TILE, D), lambda i: (i, 0)),
            pl.BlockSpec((TOKEN_TILE, 8), lambda i: (i, 0)),
            pl.BlockSpec((2 * TOKEN_TILE, D), lambda i: (i, 0)),
        ],
        out_specs=pl.BlockSpec((TOKEN_TILE, D), lambda i: (i, 0)),
        out_shape=jax.ShapeDtypeStruct((B, D), jnp.float32),
        compiler_params=pltpu.CompilerParams(
            dimension_semantics=("arbitrary",)),
    )(encoded, wq, o_tok)


@jax.jit
def kernel(x, W_enc, b_enc, W_gate, U, V, gamma):
    encoded, wq, idx = _encode_route(x, W_enc, b_enc, W_gate, gamma)
    slot_token, slot_out, tile_expert = _dispatch_plan(idx)
    x_sorted = _sc_gather(encoded, slot_token)
    Vt = V.transpose(0, 2, 1)
    o_sorted = _experts(x_sorted, tile_expert, U, Vt)
    o_tok = _sc_scatter(o_sorted, slot_out)
    return _combine(encoded, wq, o_tok)


# speedup vs baseline: 4.1693x; 4.1693x over previous
import jax
import jax.numpy as jnp
from jax.experimental import pallas as pl
from jax.experimental.pallas import tpu as pltpu

B = 8192
D = 2048
E = 16
R = 128
TOKEN_TILE = 512
NEG_BIG = -3.0e38


def _bf16_dot(a_f32, w_bf16):
    return jax.lax.dot_general(
        a_f32.astype(jnp.bfloat16), w_bf16, (((1,), (1,)), ((), ())),
        preferred_element_type=jnp.float32)


def _encode_route_body(x_ref, wenc_ref, benc_ref, wgate_ref, enc_ref,
                       comb_ref):
    enc = _bf16_dot(x_ref[...], wenc_ref[...]) + benc_ref[...]
    enc_ref[...] = enc

    logits = _bf16_dot(enc, wgate_ref[...])

    lane = jax.lax.broadcasted_iota(jnp.int32, logits.shape, 1)
    v0 = jnp.max(logits, axis=1, keepdims=True)
    i0 = jnp.min(jnp.where(logits == v0, lane, E), axis=1, keepdims=True)
    masked = jnp.where(lane == i0, NEG_BIG, logits)
    v1 = jnp.max(masked, axis=1, keepdims=True)
    i1 = jnp.min(jnp.where(masked == v1, lane, E), axis=1, keepdims=True)

    e1 = jnp.exp(v1 - v0)
    denom = 1.0 + e1 + 1e-12
    w0 = 1.0 / denom
    w1 = e1 / denom
    w0 = jnp.where(w0 > 1e-12, w0, 0.0)
    w1 = jnp.where(w1 > 1e-12, w1, 0.0)

    comb_ref[...] = (jnp.where(lane == i0, w0, 0.0)
                     + jnp.where(lane == i1, w1, 0.0))


def _experts_body(enc_ref, comb_ref, gamma_ref, u_ref, vt_ref, y_ref):
    enc = enc_ref[...]
    comb = comb_ref[...]
    comb_g = comb * gamma_ref[...]
    h = _bf16_dot(enc, u_ref[...])
    h = h * jax.nn.sigmoid(h)
    h = jnp.concatenate(
        [h[:, m * R:(m + 1) * R] * comb_g[:, m:m + 1] for m in range(E)],
        axis=1)
    o = jax.lax.dot_general(
        h.astype(jnp.bfloat16), vt_ref[...], (((1,), (0,)), ((), ())),
        preferred_element_type=jnp.float32)
    y_ref[...] = enc * jnp.sum(comb, axis=1, keepdims=True) + o


@jax.jit
def kernel(x, W_enc, b_enc, W_gate, U, V, gamma):
    bf16 = jnp.bfloat16
    wenc_b = W_enc.astype(bf16)
    wgate_b = W_gate.astype(bf16)
    u_b = U.reshape(E * R, D).astype(bf16)
    vt_b = V.transpose(0, 2, 1).reshape(E * R, D).astype(bf16)

    grid = (B // TOKEN_TILE,)
    encoded, combine = pl.pallas_call(
        _encode_route_body,
        grid=grid,
        in_specs=[
            pl.BlockSpec((TOKEN_TILE, D), lambda i: (i, 0)),
            pl.BlockSpec((D, D), lambda i: (0, 0)),
            pl.BlockSpec((1, D), lambda i: (0, 0)),
            pl.BlockSpec((E, D), lambda i: (0, 0)),
        ],
        out_specs=[
            pl.BlockSpec((TOKEN_TILE, D), lambda i: (i, 0)),
            pl.BlockSpec((TOKEN_TILE, E), lambda i: (i, 0)),
        ],
        out_shape=[
            jax.ShapeDtypeStruct((B, D), jnp.float32),
            jax.ShapeDtypeStruct((B, E), jnp.float32),
        ],
        compiler_params=pltpu.CompilerParams(
            dimension_semantics=("arbitrary",),
        ),
    )(x, wenc_b, b_enc.reshape(1, D), wgate_b)

    y = pl.pallas_call(
        _experts_body,
        grid=grid,
        in_specs=[
            pl.BlockSpec((TOKEN_TILE, D), lambda i: (i, 0)),
            pl.BlockSpec((TOKEN_TILE, E), lambda i: (i, 0)),
            pl.BlockSpec((1, E), lambda i: (0, 0)),
            pl.BlockSpec((E * R, D), lambda i: (0, 0)),
            pl.BlockSpec((E * R, D), lambda i: (0, 0)),
        ],
        out_specs=pl.BlockSpec((TOKEN_TILE, D), lambda i: (i, 0)),
        out_shape=jax.ShapeDtypeStruct((B, D), jnp.float32),
        compiler_params=pltpu.CompilerParams(
            dimension_semantics=("arbitrary",),
        ),
    )(encoded, combine, gamma.reshape(1, E), u_b, vt_b)
    return y
